# baseline (speedup 1.0000x reference)
import functools
import math

import numpy as np
import jax
import jax.numpy as jnp
from jax.experimental import pallas as pl
from jax.experimental.pallas import tpu as pltpu
from jax.experimental.pallas import tpu_sc as plsc

_NUM_EMB = 1000000
_D = 64
_SEQ = 200
_B = 4096
_HALF = _SEQ // 2


def _pe_table():
    position = np.arange(_SEQ, dtype=np.float32)[:, None]
    div = np.exp(
        np.arange(0, _D, 2, dtype=np.float32) * (-math.log(10000.0) / _D)
    )
    pe = np.zeros((_SEQ, _D), dtype=np.float32)
    pe[:, 0::2] = np.sin(position * div)
    pe[:, 1::2] = np.cos(position * div)
    return pe


_MESH = plsc.VectorSubcoreMesh(core_axis_name="c", subcore_axis_name="s")


def kernel(data, table):
    pe = jnp.asarray(_pe_table())
    idx3 = data.reshape(_B, 2, _HALF).astype(jnp.int32)

    @functools.partial(
        pl.kernel,
        out_type=jax.ShapeDtypeStruct((_B * _SEQ, _D), jnp.float32),
        mesh=_MESH,
        scratch_types=[
            pltpu.VMEM((_SEQ, _D), jnp.float32),
            pltpu.SemaphoreType.DMA,
        ],
        compiler_params=pltpu.CompilerParams(use_tc_tiling_on_sc=False),
    )
    def _emb(idx_hbm, pe_hbm, table_hbm, out_hbm, pe_v, sem):
        pltpu.async_copy(pe_hbm, pe_v, sem).wait()

        def body(i_vmem, o_vmem):
            cp0 = pltpu.async_copy(
                table_hbm.at[i_vmem.at[0, 0]], o_vmem.at[pl.ds(0, _HALF)], sem
            )
            cp1 = pltpu.async_copy(
                table_hbm.at[i_vmem.at[0, 1]],
                o_vmem.at[pl.ds(_HALF, _HALF)],
                sem,
            )
            cp0.wait()
            cp1.wait()

            @pl.loop(0, _SEQ)
            def _(r):
                for c in range(_D // 16):
                    sl = pl.ds(c * 16, 16)
                    o_vmem[r, sl] = o_vmem[r, sl] + pe_v[r, sl]

        pltpu.emit_pipeline(
            body,
            grid=(_B,),
            in_specs=[
                pl.BlockSpec((1, 2, _HALF), lambda i: (i, 0, 0)),
            ],
            out_specs=[
                pl.BlockSpec((_SEQ, _D), lambda i: (i, 0)),
            ],
            core_axis_name=("c", "s"),
            dimension_semantics=(pltpu.PARALLEL,),
        )(idx_hbm, out_hbm)

    out = _emb(idx3, pe, table)
    return out.reshape(_B, _SEQ, _D)


# speedup vs baseline: 1.4181x; 1.4181x over previous
import functools
import math

import numpy as np
import jax
import jax.numpy as jnp
from jax import lax
from jax.experimental import pallas as pl
from jax.experimental.pallas import tpu as pltpu
from jax.experimental.pallas import tpu_sc as plsc

_NUM_EMB = 1000000
_D = 64
_SEQ = 200
_B = 4096
_S0 = 104
_S1 = _SEQ - _S0

_NW = 32
_RPT = _B // _NW
_IPT = _RPT * _SEQ
_R = 4
_T = _RPT // _R


def _pe_table():
    position = np.arange(_SEQ, dtype=np.float32)[:, None]
    div = np.exp(
        np.arange(0, _D, 2, dtype=np.float32) * (-math.log(10000.0) / _D)
    )
    pe = np.zeros((_SEQ, _D), dtype=np.float32)
    pe[:, 0::2] = np.sin(position * div)
    pe[:, 1::2] = np.cos(position * div)
    return pe


_MESH = plsc.VectorSubcoreMesh(core_axis_name="c", subcore_axis_name="s")


def kernel(data, table):
    pe = jnp.asarray(_pe_table())
    idx_flat = data.reshape(_B * _SEQ).astype(jnp.int32)

    @functools.partial(
        pl.kernel,
        out_type=jax.ShapeDtypeStruct((_B * _SEQ, _D), jnp.float32),
        mesh=_MESH,
        scratch_types=[
            pltpu.VMEM((_IPT,), jnp.int32),
            pltpu.VMEM((_SEQ, _D), jnp.float32),
            pltpu.VMEM((_R, _SEQ, _D), jnp.float32),
            pltpu.SemaphoreType.DMA((_R,)),
            pltpu.SemaphoreType.DMA((_R,)),
            pltpu.SemaphoreType.DMA,
        ],
        compiler_params=pltpu.CompilerParams(use_tc_tiling_on_sc=False),
    )
    def _emb(idx_hbm, pe_hbm, table_hbm, out_hbm, idx_v, pe_v, bufs, gsem,
             osem, psem):
        wid = lax.axis_index("s") * 2 + lax.axis_index("c")
        ibase = wid * _IPT
        rbase = wid * _RPT

        cp_i = pltpu.async_copy(idx_hbm.at[pl.ds(ibase, _IPT)], idx_v, psem)
        cp_p = pltpu.async_copy(pe_hbm, pe_v, psem)
        cp_i.wait()
        cp_p.wait()

        def gather_cps(n, k):
            buf = bufs.at[k]
            c0 = pltpu.make_async_copy(
                table_hbm.at[idx_v.at[pl.ds(n * _SEQ, _S0)]],
                buf.at[pl.ds(0, _S0)],
                gsem.at[k],
            )
            c1 = pltpu.make_async_copy(
                table_hbm.at[idx_v.at[pl.ds(n * _SEQ + _S0, _S1)]],
                buf.at[pl.ds(_S0, _S1)],
                gsem.at[k],
            )
            return c0, c1

        def out_cp(m, k):
            return pltpu.make_async_copy(
                bufs.at[k],
                out_hbm.at[pl.ds((rbase + m) * _SEQ, _SEQ)],
                osem.at[k],
            )

        def process(m, k):
            c0, c1 = gather_cps(m, k)
            c0.wait()
            c1.wait()
            buf = bufs.at[k]

            @pl.loop(0, _SEQ, step=8)
            def _(r0):
                for dr in range(8):
                    r = r0 + dr
                    for c in range(_D // 16):
                        sl = pl.ds(c * 16, 16)
                        buf[r, sl] = buf[r, sl] + pe_v[r, sl]

            out_cp(m, k).start()

        @pl.loop(0, _T)
        def _(t):
            for k in range(_R):
                n = t * _R + k

                @pl.when(t >= 1)
                def _():
                    out_cp(n - _R, k).wait()

                c0, c1 = gather_cps(n, k)
                c0.start()
                c1.start()

                k2 = (k + 1) % _R
                if k == _R - 1:
                    process(n - (_R - 1), k2)
                else:

                    @pl.when(t >= 1)
                    def _():
                        process(n - (_R - 1), k2)

        last = _R * _T
        for j in range(_R - 1):
            m = last - (_R - 1) + j
            process(m, m % _R)
        for k in range(_R):
            out_cp(last - _R + k, k).wait()

    out = _emb(idx_flat, pe, table)
    return out.reshape(_B, _SEQ, _D)


# speedup vs baseline: 1.4187x; 1.0004x over previous
import functools
import math

import numpy as np
import jax
import jax.numpy as jnp
from jax import lax
from jax.experimental import pallas as pl
from jax.experimental.pallas import tpu as pltpu
from jax.experimental.pallas import tpu_sc as plsc

_NUM_EMB = 1000000
_D = 64
_SEQ = 200
_B = 4096
_S0 = 104
_S1 = _SEQ - _S0

_NW = 32
_RPT = _B // _NW
_IPT = _RPT * _SEQ
_R = 4
_T = _RPT // _R


def _pe_table():
    position = np.arange(_SEQ, dtype=np.float32)[:, None]
    div = np.exp(
        np.arange(0, _D, 2, dtype=np.float32) * (-math.log(10000.0) / _D)
    )
    pe = np.zeros((_SEQ, _D), dtype=np.float32)
    pe[:, 0::2] = np.sin(position * div)
    pe[:, 1::2] = np.cos(position * div)
    return pe


_MESH = plsc.VectorSubcoreMesh(core_axis_name="c", subcore_axis_name="s")


def kernel(data, table):
    pe = jnp.asarray(_pe_table())
    idx = data.astype(jnp.int32)

    @functools.partial(
        pl.kernel,
        out_type=jax.ShapeDtypeStruct((_B, _SEQ, _D), jnp.float32),
        mesh=_MESH,
        scratch_types=[
            pltpu.VMEM((_RPT, _SEQ), jnp.int32),
            pltpu.VMEM((_SEQ, _D), jnp.float32),
            pltpu.VMEM((_R, _SEQ, _D), jnp.float32),
            pltpu.SemaphoreType.DMA((_R,)),
            pltpu.SemaphoreType.DMA((_R,)),
            pltpu.SemaphoreType.DMA,
        ],
        compiler_params=pltpu.CompilerParams(use_tc_tiling_on_sc=False),
    )
    def _emb(idx_hbm, pe_hbm, table_hbm, out_hbm, idx_v, pe_v, bufs, gsem,
             osem, psem):
        wid = lax.axis_index("s") * 2 + lax.axis_index("c")
        rbase = wid * _RPT

        cp_i = pltpu.async_copy(idx_hbm.at[pl.ds(rbase, _RPT)], idx_v, psem)
        cp_p = pltpu.async_copy(pe_hbm, pe_v, psem)
        cp_i.wait()
        cp_p.wait()

        def gather_cps(n, k):
            buf = bufs.at[k]
            c0 = pltpu.make_async_copy(
                table_hbm.at[idx_v.at[n, pl.ds(0, _S0)]],
                buf.at[pl.ds(0, _S0)],
                gsem.at[k],
            )
            c1 = pltpu.make_async_copy(
                table_hbm.at[idx_v.at[n, pl.ds(_S0, _S1)]],
                buf.at[pl.ds(_S0, _S1)],
                gsem.at[k],
            )
            return c0, c1

        def out_cp(m, k):
            return pltpu.make_async_copy(
                bufs.at[k],
                out_hbm.at[rbase + m],
                osem.at[k],
            )

        def process(m, k):
            c0, c1 = gather_cps(m, k)
            c0.wait()
            c1.wait()
            buf = bufs.at[k]

            @pl.loop(0, _SEQ, step=8)
            def _(r0):
                for dr in range(8):
                    r = r0 + dr
                    for c in range(_D // 16):
                        sl = pl.ds(c * 16, 16)
                        buf[r, sl] = buf[r, sl] + pe_v[r, sl]

            out_cp(m, k).start()

        @pl.loop(0, _T)
        def _(t):
            for k in range(_R):
                n = t * _R + k

                @pl.when(t >= 1)
                def _():
                    out_cp(n - _R, k).wait()

                c0, c1 = gather_cps(n, k)
                c0.start()
                c1.start()

                k2 = (k + 1) % _R
                if k == _R - 1:
                    process(n - (_R - 1), k2)
                else:

                    @pl.when(t >= 1)
                    def _():
                        process(n - (_R - 1), k2)

        last = _R * _T
        for j in range(_R - 1):
            m = last - (_R - 1) + j
            process(m, m % _R)
        for k in range(_R):
            out_cp(last - _R + k, k).wait()

    return _emb(idx, pe, table)
